# baseline (speedup 1.0000x reference)
import jax
import jax.numpy as jnp
from jax import lax
from jax.experimental import pallas as pl

_B, _S, _V = 1024, 50, 999
_ROWS = 16


def _onehot_body(x_ref, o_ref):
    idx = x_ref[...]
    cols = lax.broadcasted_iota(jnp.int32, (_ROWS, _S, _V), 2)
    o_ref[...] = (cols + 1 == idx[:, :, None]).astype(jnp.float32)


def kernel(x):
    grid = (_B // _ROWS,)
    return pl.pallas_call(
        _onehot_body,
        grid=grid,
        in_specs=[pl.BlockSpec((_ROWS, _S), lambda i: (i, 0))],
        out_specs=pl.BlockSpec((_ROWS, _S, _V), lambda i: (i, 0, 0)),
        out_shape=jax.ShapeDtypeStruct((_B, _S, _V), jnp.float32),
    )(x)
